# Initial kernel scaffold; baseline (speedup 1.0000x reference)
import jax
import jax.numpy as jnp
from jax.experimental import pallas as pl


def kernel(out, z, batch):
    raise NotImplementedError("write your pallas kernel here")


# speedup vs baseline: 9.1112x; 9.1112x over previous
import functools

import jax
import jax.numpy as jnp
from jax import lax
from jax.experimental import pallas as pl
from jax.experimental.pallas import tpu as pltpu
from jax.experimental.pallas import tpu_sc as plsc

N = 100000
D = 128
G = 512
S = 2 * G
NC, NS = 2, 16
NW = NC * NS
L = 16
GRP = 128
FULL = N // GRP
TAIL = N - FULL * GRP
K = (FULL + NW - 1) // NW
TAIL_WID = FULL % NW
ZR = S // NS
ZB = 16


def _sc_body(out_hbm, z_hbm, b_hbm, part_hbm, cnt_hbm,
             acc, rows_v, zb_v, zt_v, idx_v, idxt_v, cnt_v, zero_v):
    cid = lax.axis_index("c")
    sid = lax.axis_index("s")
    wid = sid * NC + cid
    lanes = lax.iota(jnp.int32, L)
    ones = jnp.ones((L,), jnp.float32)

    def _fill_zero(i, _):
        for j in range(D // L):
            zero_v[i, pl.ds(j * L, L)] = jnp.zeros((L,), jnp.float32)
        return 0
    lax.fori_loop(0, ZB, _fill_zero, 0)

    def _fill_cnt(i, _):
        for r in range(L):
            cnt_v[r, pl.ds(i * L, L)] = jnp.zeros((L,), jnp.float32)
        return 0
    lax.fori_loop(0, S // L, _fill_cnt, 0)

    for t in range(ZR // ZB):
        pltpu.sync_copy(zero_v, acc.at[pl.ds(sid * ZR + t * ZB, ZB)])
    plsc.subcore_barrier()

    def _index_chunk(dst, zsrc, bsrc, j):
        zz = zsrc[pl.ds(j * L, L)]
        bb = bsrc[pl.ds(j * L, L)]
        idx = bb + jnp.where(zz == jnp.int32(100), jnp.int32(G),
                             jnp.int32(0))
        dst[0, pl.ds(j * L, L)] = idx
        plsc.addupdate_scatter(cnt_v, [lanes, idx], ones)

    def _group(k, _):
        g = wid + NW * k

        @pl.when(g < FULL)
        def _():
            base = pl.multiple_of(g * GRP, GRP)
            pltpu.sync_copy(z_hbm.at[pl.ds(base, GRP)], zb_v.at[0])
            pltpu.sync_copy(b_hbm.at[pl.ds(base, GRP)], zb_v.at[1])
            pltpu.sync_copy(out_hbm.at[pl.ds(base, GRP)], rows_v)
            for j in range(GRP // L):
                _index_chunk(idx_v, zb_v.at[0], zb_v.at[1], j)
            pltpu.sync_copy(rows_v, acc.at[idx_v.at[0]], add=True)
        return 0
    lax.fori_loop(0, K, _group, 0)

    if TAIL:
        @pl.when(wid == TAIL_WID)
        def _():
            base = FULL * GRP
            pltpu.sync_copy(z_hbm.at[pl.ds(base, TAIL)], zt_v.at[0])
            pltpu.sync_copy(b_hbm.at[pl.ds(base, TAIL)], zt_v.at[1])
            pltpu.sync_copy(out_hbm.at[pl.ds(base, TAIL)],
                            rows_v.at[pl.ds(0, TAIL)])
            for j in range(TAIL // L):
                _index_chunk(idxt_v, zt_v.at[0], zt_v.at[1], j)
            pltpu.sync_copy(rows_v.at[pl.ds(0, TAIL)],
                            acc.at[idxt_v.at[0]], add=True)

    plsc.subcore_barrier()
    pltpu.sync_copy(acc.at[pl.ds(sid * ZR, ZR)],
                    part_hbm.at[cid, pl.ds(sid * ZR, ZR)])
    pltpu.sync_copy(cnt_v, cnt_hbm.at[wid])


_sc_pool = functools.partial(
    pl.kernel,
    out_type=(jax.ShapeDtypeStruct((NC, S, D), jnp.float32),
              jax.ShapeDtypeStruct((NW, L, S), jnp.float32)),
    mesh=plsc.VectorSubcoreMesh(core_axis_name="c", subcore_axis_name="s",
                                num_cores=NC, num_subcores=NS),
    compiler_params=pltpu.CompilerParams(needs_layout_passes=False),
    scratch_types=[
        pltpu.VMEM_SHARED((S, D), jnp.float32),
        pltpu.VMEM((GRP, D), jnp.float32),
        pltpu.VMEM((2, GRP), jnp.int32),
        pltpu.VMEM((2, max(TAIL, L)), jnp.int32),
        pltpu.VMEM((1, GRP), jnp.int32),
        pltpu.VMEM((1, max(TAIL, L)), jnp.int32),
        pltpu.VMEM((L, S), jnp.float32),
        pltpu.VMEM((ZB, D), jnp.float32),
    ],
)(_sc_body)


def _fin_body(p_ref, c_ref, o_ref):
    sums = p_ref[0] + p_ref[1]
    cnts = jnp.sum(c_ref[...], axis=(0, 1))
    denom = jnp.maximum(cnts, 1.0)[:, None]
    means = sums / denom
    o_ref[:, :D] = means[:G]
    o_ref[:, D:] = means[G:]


def kernel(out, z, batch):
    part, cnts = _sc_pool(out, z.astype(jnp.int32), batch.astype(jnp.int32))
    return pl.pallas_call(
        _fin_body,
        out_shape=jax.ShapeDtypeStruct((G, 2 * D), jnp.float32),
    )(part, cnts)


# speedup vs baseline: 15.7333x; 1.7268x over previous
import functools

import jax
import jax.numpy as jnp
from jax import lax
from jax.experimental import pallas as pl
from jax.experimental.pallas import tpu as pltpu
from jax.experimental.pallas import tpu_sc as plsc

N = 100000
D = 128
G = 512
S = 2 * G
TRASH = S
S2 = S + 128
NC, NS = 2, 16
NW = NC * NS
L = 16
GRP = 128
FULL = N // GRP
TAIL = N - FULL * GRP
LASTBASE = N - GRP
VS_TAIL = GRP - TAIL
NGRP = FULL + (1 if TAIL else 0)
K = (NGRP + NW - 1) // NW
KP = (K + 1) // 2
ZR = S2 // NS
ZB = 24


def _sc_body(out_hbm, z_hbm, b_hbm, part_hbm, cnt_hbm,
             acc, rows0, rows1, zb0, zb1, idx_v, cnt_v, zero_v,
             sem0, sem1):
    cid = lax.axis_index("c")
    sid = lax.axis_index("s")
    wid = sid * NC + cid
    lanes = lax.iota(jnp.int32, L)
    ones = jnp.ones((L,), jnp.float32)

    def _fill_zero(i, _):
        for j in range(D // L):
            zero_v[i, pl.ds(j * L, L)] = jnp.zeros((L,), jnp.float32)
        return 0
    lax.fori_loop(0, ZB, _fill_zero, 0)

    def _fill_cnt(i, _):
        for r in range(L):
            cnt_v[r, pl.ds(i * L, L)] = jnp.zeros((L,), jnp.float32)
        return 0
    lax.fori_loop(0, S2 // L, _fill_cnt, 0)

    for t in range(ZR // ZB):
        pltpu.sync_copy(zero_v, acc.at[pl.ds(sid * ZR + t * ZB, ZB)])
    plsc.subcore_barrier()

    def _base(k):
        g = wid + NW * k
        return g, pl.multiple_of(
            jnp.where(g < FULL, g * GRP, jnp.int32(LASTBASE)), 8)

    def _start(k, zb_b, rows_b, sem_b):
        _, base = _base(k)
        pltpu.async_copy(z_hbm.at[pl.ds(base, GRP)], zb_b.at[0], sem_b)
        pltpu.async_copy(b_hbm.at[pl.ds(base, GRP)], zb_b.at[1], sem_b)
        pltpu.async_copy(out_hbm.at[pl.ds(base, GRP)], rows_b, sem_b)

    def _process(k, zb_b, rows_b, sem_b):
        g, _ = _base(k)
        vs = jnp.where(g < FULL, jnp.int32(0),
                       jnp.where(g == FULL, jnp.int32(VS_TAIL),
                                 jnp.int32(GRP)))
        pltpu.make_async_copy(z_hbm.at[pl.ds(0, GRP)], zb_b.at[0],
                              sem_b).wait()
        pltpu.make_async_copy(b_hbm.at[pl.ds(0, GRP)], zb_b.at[1],
                              sem_b).wait()
        pltpu.make_async_copy(out_hbm.at[pl.ds(0, GRP)], rows_b,
                              sem_b).wait()
        for j in range(GRP // L):
            zz = zb_b[0, pl.ds(j * L, L)]
            bb = zb_b[1, pl.ds(j * L, L)]
            slot = bb + jnp.where(zz == jnp.int32(100), jnp.int32(G),
                                  jnp.int32(0))
            idx = jnp.where(j * L + lanes >= vs, slot, jnp.int32(TRASH))
            idx_v[0, pl.ds(j * L, L)] = idx
            plsc.addupdate_scatter(cnt_v, [lanes, idx], ones)
        pltpu.sync_copy(rows_b, acc.at[idx_v.at[0]], add=True)

    _start(0, zb0, rows0, sem0)
    _start(1, zb1, rows1, sem1)

    def _pair(kk, _):
        k0 = 2 * kk
        k1 = k0 + 1
        _process(k0, zb0, rows0, sem0)

        @pl.when(k0 + 2 < K)
        def _():
            _start(k0 + 2, zb0, rows0, sem0)

        @pl.when(k1 < K)
        def _():
            _process(k1, zb1, rows1, sem1)

        @pl.when(k1 + 2 < K)
        def _():
            _start(k1 + 2, zb1, rows1, sem1)
        return 0
    lax.fori_loop(0, KP, _pair, 0)

    plsc.subcore_barrier()
    pltpu.sync_copy(acc.at[pl.ds(sid * ZR, ZR)],
                    part_hbm.at[cid, pl.ds(sid * ZR, ZR)])
    pltpu.sync_copy(cnt_v, cnt_hbm.at[wid])


_sc_pool = functools.partial(
    pl.kernel,
    out_type=(jax.ShapeDtypeStruct((NC, S2, D), jnp.float32),
              jax.ShapeDtypeStruct((NW, L, S2), jnp.float32)),
    mesh=plsc.VectorSubcoreMesh(core_axis_name="c", subcore_axis_name="s",
                                num_cores=NC, num_subcores=NS),
    compiler_params=pltpu.CompilerParams(needs_layout_passes=False),
    scratch_types=[
        pltpu.VMEM_SHARED((S2, D), jnp.float32),
        pltpu.VMEM((GRP, D), jnp.float32),
        pltpu.VMEM((GRP, D), jnp.float32),
        pltpu.VMEM((2, GRP), jnp.int32),
        pltpu.VMEM((2, GRP), jnp.int32),
        pltpu.VMEM((1, GRP), jnp.int32),
        pltpu.VMEM((L, S2), jnp.float32),
        pltpu.VMEM((ZB, D), jnp.float32),
        pltpu.SemaphoreType.DMA,
        pltpu.SemaphoreType.DMA,
    ],
)(_sc_body)


def _fin_body(p_ref, c_ref, o_ref):
    sums = p_ref[0] + p_ref[1]
    cnts = jnp.sum(c_ref[...], axis=(0, 1))
    denom = jnp.maximum(cnts, 1.0)[:, None]
    means = sums / denom
    o_ref[:, :D] = means[:G]
    o_ref[:, D:] = means[G:S]


def kernel(out, z, batch):
    part, cnts = _sc_pool(out, z.astype(jnp.int32), batch.astype(jnp.int32))
    return pl.pallas_call(
        _fin_body,
        out_shape=jax.ShapeDtypeStruct((G, 2 * D), jnp.float32),
    )(part, cnts)
